# Initial kernel scaffold; baseline (speedup 1.0000x reference)
import jax
import jax.numpy as jnp
from jax.experimental import pallas as pl


def kernel(xyz, w1, b1, gamma, beta, w2, b2):
    raise NotImplementedError("write your pallas kernel here")


# speedup vs baseline: 3.3719x; 3.3719x over previous
import functools

import jax
import jax.numpy as jnp
from jax.experimental import pallas as pl
from jax.experimental.pallas import tpu as pltpu


def _moments_kernel(x_ref, mom_ref):
    x = x_ref[...]
    cin = x.shape[1]
    s = jnp.sum(jnp.sum(x, axis=2, keepdims=True), axis=0)
    cols = [s]
    for c in range(cin):
        p = jnp.sum(x * x[:, c:c + 1, :], axis=2, keepdims=True)
        cols.append(jnp.sum(p, axis=0))
    mom_ref[...] = jnp.concatenate(cols, axis=1)[None]


def _apply_kernel(x_ref, w1_ref, w2_ref, p_ref, o_ref):
    x = x_ref[...]
    w1 = w1_ref[...]
    cin = x.shape[0]

    h = w1[:, 0:1] * x[0:1, :]
    for c in range(1, cin):
        h = h + w1[:, c:c + 1] * x[c:c + 1, :]

    r = jnp.maximum(h + p_ref[:, 0:1], 0.0).astype(jnp.bfloat16)

    out = jnp.dot(w2_ref[...], r,
                  preferred_element_type=jnp.float32) + p_ref[:, 1:2]
    o_ref[...] = out


def _position_embedding(xyz, w1, gamma, beta, w2, b2, *, tile_n=4096,
                        eps=1e-5):
    B, N, Cin = xyz.shape
    F = w1.shape[0]

    x_t = jnp.transpose(xyz.astype(jnp.float32), (0, 2, 1))

    if N <= tile_n:
        tn, n_pad = N, N
    else:
        tn = tile_n
        n_pad = ((N + tn - 1) // tn) * tn
    if n_pad != N:
        x_t = jnp.pad(x_t, ((0, 0), (0, 0), (0, n_pad - N)))
    n_tiles = n_pad // tn

    bb = next(d for d in (8, 4, 2, 1) if B % d == 0)
    g = B // bb
    parts = pl.pallas_call(
        _moments_kernel,
        out_shape=jax.ShapeDtypeStruct((g, Cin, Cin + 1), jnp.float32),
        grid=(g,),
        in_specs=[pl.BlockSpec((bb, Cin, n_pad), lambda i: (i, 0, 0))],
        out_specs=pl.BlockSpec((1, Cin, Cin + 1), lambda i: (i, 0, 0)),
        compiler_params=pltpu.CompilerParams(
            dimension_semantics=("parallel",),
            vmem_limit_bytes=64 * 1024 * 1024),
    )(x_t)
    mom = jnp.sum(parts, axis=0)

    P = B * N
    w1f = w1.astype(jnp.float32)
    mean_x = mom[:, 0] / P
    e_xxt = mom[:, 1:] / P
    cov_x = e_xxt - jnp.outer(mean_x, mean_x)
    mean_h = w1f @ mean_x
    var_h = jnp.maximum(jnp.einsum("fc,cd,fd->f", w1f, cov_x, w1f), 0.0)
    scale = gamma.astype(jnp.float32) * jax.lax.rsqrt(var_h + eps)
    shift = beta.astype(jnp.float32) - mean_h * scale
    w1s = w1f * scale[:, None]
    pvec = jnp.stack([shift, b2.astype(jnp.float32)], axis=1)

    out = pl.pallas_call(
        _apply_kernel,
        out_shape=jax.ShapeDtypeStruct((B, F, n_pad), jnp.float32),
        grid=(B, n_tiles),
        in_specs=[
            pl.BlockSpec((None, Cin, tn), lambda b, t: (b, 0, t)),
            pl.BlockSpec((F, Cin), lambda b, t: (0, 0)),
            pl.BlockSpec((F, F), lambda b, t: (0, 0)),
            pl.BlockSpec((F, 2), lambda b, t: (0, 0)),
        ],
        out_specs=pl.BlockSpec((None, F, tn), lambda b, t: (b, 0, t)),
        compiler_params=pltpu.CompilerParams(
            dimension_semantics=("parallel", "parallel"),
            vmem_limit_bytes=64 * 1024 * 1024),
    )(x_t, w1s, w2.astype(jnp.bfloat16), pvec)

    if n_pad != N:
        out = out[:, :, :N]
    return out


def kernel(xyz, w1, b1, gamma, beta, w2, b2):
    del b1
    return _position_embedding(xyz, w1, gamma, beta, w2, b2, tile_n=4096)


# speedup vs baseline: 4.9590x; 1.4707x over previous
import functools

import jax
import jax.numpy as jnp
from jax.experimental import pallas as pl
from jax.experimental.pallas import tpu as pltpu


def _moments_kernel(x_ref, mom_ref):
    cin = x_ref.shape[1] - 1
    x = x_ref[:, :cin, :]
    s = jnp.sum(jnp.sum(x, axis=2, keepdims=True), axis=0)
    cols = [s]
    for c in range(cin):
        p = jnp.sum(x * x[:, c:c + 1, :], axis=2, keepdims=True)
        cols.append(jnp.sum(p, axis=0))
    mom_ref[...] = jnp.concatenate(cols, axis=1)[None]


def _apply_kernel(x_ref, w1_ref, w2_ref, o_ref):
    for i in range(x_ref.shape[0]):
        x16 = x_ref[i].astype(jnp.bfloat16)
        h = jnp.dot(w1_ref[...], x16,
                    preferred_element_type=jnp.float32)
        r = jnp.maximum(h, 0.0).astype(jnp.bfloat16)
        ones = jnp.ones((1, r.shape[1]), jnp.bfloat16)
        r_aug = jnp.concatenate([r, ones], axis=0)
        out = jnp.dot(w2_ref[...], r_aug,
                      preferred_element_type=jnp.float32)
        o_ref[i] = out


def _position_embedding(xyz, w1, gamma, beta, w2, b2, *, tile_n=4096,
                        eps=1e-5):
    B, N, Cin = xyz.shape
    F = w1.shape[0]

    x_t = jnp.transpose(xyz.astype(jnp.float32), (0, 2, 1))

    if N <= tile_n:
        tn, n_pad = N, N
    else:
        tn = tile_n
        n_pad = ((N + tn - 1) // tn) * tn
    if n_pad != N:
        x_t = jnp.pad(x_t, ((0, 0), (0, 0), (0, n_pad - N)))
    n_tiles = n_pad // tn
    x_t4 = jnp.concatenate(
        [x_t, jnp.ones((B, 1, n_pad), jnp.float32)], axis=1)

    bb = next(d for d in (8, 4, 2, 1) if B % d == 0)
    g = B // bb
    parts = pl.pallas_call(
        _moments_kernel,
        out_shape=jax.ShapeDtypeStruct((g, Cin, Cin + 1), jnp.float32),
        grid=(g,),
        in_specs=[pl.BlockSpec((bb, Cin + 1, n_pad), lambda i: (i, 0, 0))],
        out_specs=pl.BlockSpec((1, Cin, Cin + 1), lambda i: (i, 0, 0)),
        compiler_params=pltpu.CompilerParams(
            dimension_semantics=("parallel",),
            vmem_limit_bytes=64 * 1024 * 1024),
    )(x_t4)
    mom = jnp.sum(parts, axis=0)

    P = B * N
    w1f = w1.astype(jnp.float32)
    mean_x = mom[:, 0] / P
    e_xxt = mom[:, 1:] / P
    cov_x = e_xxt - jnp.outer(mean_x, mean_x)
    mean_h = w1f @ mean_x
    var_h = jnp.maximum(jnp.einsum("fc,cd,fd->f", w1f, cov_x, w1f), 0.0)
    scale = gamma.astype(jnp.float32) * jax.lax.rsqrt(var_h + eps)
    shift = beta.astype(jnp.float32) - mean_h * scale
    w1a = jnp.concatenate(
        [w1f * scale[:, None], shift[:, None]],
        axis=1).astype(jnp.bfloat16)
    w2b = jnp.concatenate(
        [w2.astype(jnp.float32), b2.astype(jnp.float32)[:, None]],
        axis=1).astype(jnp.bfloat16)

    bb2 = 2 if B % 2 == 0 else 1
    out = pl.pallas_call(
        _apply_kernel,
        out_shape=jax.ShapeDtypeStruct((B, F, n_pad), jnp.float32),
        grid=(B // bb2, n_tiles),
        in_specs=[
            pl.BlockSpec((bb2, Cin + 1, tn), lambda b, t: (b, 0, t)),
            pl.BlockSpec((F, Cin + 1), lambda b, t: (0, 0)),
            pl.BlockSpec((F, F + 1), lambda b, t: (0, 0)),
        ],
        out_specs=pl.BlockSpec((bb2, F, tn), lambda b, t: (b, 0, t)),
        compiler_params=pltpu.CompilerParams(
            dimension_semantics=("parallel", "parallel"),
            vmem_limit_bytes=64 * 1024 * 1024),
    )(x_t4, w1a, w2b)

    if n_pad != N:
        out = out[:, :, :N]
    return out


def kernel(xyz, w1, b1, gamma, beta, w2, b2):
    del b1
    return _position_embedding(xyz, w1, gamma, beta, w2, b2, tile_n=8192)


# speedup vs baseline: 5.0634x; 1.0210x over previous
import functools

import jax
import jax.numpy as jnp
from jax.experimental import pallas as pl
from jax.experimental.pallas import tpu as pltpu


def _moments_kernel(x_ref, mom_ref):
    cin = x_ref.shape[1] - 1
    x = x_ref[:, :cin, :]
    s = jnp.sum(jnp.sum(x, axis=2, keepdims=True), axis=0)
    cols = [s]
    for c in range(cin):
        p = jnp.sum(x * x[:, c:c + 1, :], axis=2, keepdims=True)
        cols.append(jnp.sum(p, axis=0))
    mom_ref[...] = jnp.concatenate(cols, axis=1)[None]


def _apply_kernel(x_ref, w1_ref, w2_ref, o_ref):
    w1 = w1_ref[...]
    w2 = w2_ref[...]
    tn = x_ref.shape[2]
    ch = min(4096, tn)
    for i in range(x_ref.shape[0]):
        for j in range(tn // ch):
            sl = slice(j * ch, (j + 1) * ch)
            x16 = x_ref[i, :, sl].astype(jnp.bfloat16)
            h = jnp.dot(w1, x16,
                        preferred_element_type=jnp.float32)
            r = jnp.maximum(h, 0.0).astype(jnp.bfloat16)
            ones = jnp.ones((1, ch), jnp.bfloat16)
            r_aug = jnp.concatenate([r, ones], axis=0)
            o_ref[i, :, sl] = jnp.dot(w2, r_aug,
                                      preferred_element_type=jnp.float32)


def _position_embedding(xyz, w1, gamma, beta, w2, b2, *, tile_n=4096,
                        eps=1e-5):
    B, N, Cin = xyz.shape
    F = w1.shape[0]

    x_t = jnp.transpose(xyz.astype(jnp.float32), (0, 2, 1))

    if N <= tile_n:
        tn, n_pad = N, N
    else:
        tn = tile_n
        n_pad = ((N + tn - 1) // tn) * tn
    if n_pad != N:
        x_t = jnp.pad(x_t, ((0, 0), (0, 0), (0, n_pad - N)))
    n_tiles = n_pad // tn
    x_t4 = jnp.concatenate(
        [x_t, jnp.ones((B, 1, n_pad), jnp.float32)], axis=1)

    bb = next(d for d in (8, 4, 2, 1) if B % d == 0)
    g = B // bb
    parts = pl.pallas_call(
        _moments_kernel,
        out_shape=jax.ShapeDtypeStruct((g, Cin, Cin + 1), jnp.float32),
        grid=(g,),
        in_specs=[pl.BlockSpec((bb, Cin + 1, n_pad), lambda i: (i, 0, 0))],
        out_specs=pl.BlockSpec((1, Cin, Cin + 1), lambda i: (i, 0, 0)),
        compiler_params=pltpu.CompilerParams(
            dimension_semantics=("parallel",),
            vmem_limit_bytes=64 * 1024 * 1024),
    )(x_t4)
    mom = jnp.sum(parts, axis=0)

    P = B * N
    w1f = w1.astype(jnp.float32)
    mean_x = mom[:, 0] / P
    e_xxt = mom[:, 1:] / P
    cov_x = e_xxt - jnp.outer(mean_x, mean_x)
    mean_h = w1f @ mean_x
    var_h = jnp.maximum(jnp.einsum("fc,cd,fd->f", w1f, cov_x, w1f), 0.0)
    scale = gamma.astype(jnp.float32) * jax.lax.rsqrt(var_h + eps)
    shift = beta.astype(jnp.float32) - mean_h * scale
    w1a = jnp.concatenate(
        [w1f * scale[:, None], shift[:, None]],
        axis=1).astype(jnp.bfloat16)
    w2b = jnp.concatenate(
        [w2.astype(jnp.float32), b2.astype(jnp.float32)[:, None]],
        axis=1).astype(jnp.bfloat16)

    bb2 = next(d for d in (3, 2, 1) if B % d == 0)
    out = pl.pallas_call(
        _apply_kernel,
        out_shape=jax.ShapeDtypeStruct((B, F, n_pad), jnp.float32),
        grid=(B // bb2, n_tiles),
        in_specs=[
            pl.BlockSpec((bb2, Cin + 1, tn), lambda b, t: (b, 0, t)),
            pl.BlockSpec((F, Cin + 1), lambda b, t: (0, 0)),
            pl.BlockSpec((F, F + 1), lambda b, t: (0, 0)),
        ],
        out_specs=pl.BlockSpec((bb2, F, tn), lambda b, t: (b, 0, t)),
        compiler_params=pltpu.CompilerParams(
            dimension_semantics=("parallel", "parallel"),
            vmem_limit_bytes=64 * 1024 * 1024),
    )(x_t4, w1a, w2b)

    if n_pad != N:
        out = out[:, :, :N]
    return out


def kernel(xyz, w1, b1, gamma, beta, w2, b2):
    del b1
    return _position_embedding(xyz, w1, gamma, beta, w2, b2, tile_n=16384)


# speedup vs baseline: 5.2756x; 1.0419x over previous
import functools

import jax
import jax.numpy as jnp
from jax.experimental import pallas as pl
from jax.experimental.pallas import tpu as pltpu


def _moments_kernel(x_ref, mom_ref):
    cin = x_ref.shape[1] - 1
    x = x_ref[:, :cin, :].astype(jnp.float32)
    s = jnp.sum(jnp.sum(x, axis=2, keepdims=True), axis=0)
    cols = [s]
    for c in range(cin):
        p = jnp.sum(x * x[:, c:c + 1, :], axis=2, keepdims=True)
        cols.append(jnp.sum(p, axis=0))
    mom_ref[...] = jnp.concatenate(cols, axis=1)[None]


def _apply_kernel(x_ref, w1_ref, w2_ref, o_ref):
    for i in range(x_ref.shape[0]):
        x16 = x_ref[i]
        h = jnp.dot(w1_ref[...], x16,
                    preferred_element_type=jnp.float32)
        r = jnp.maximum(h, 0.0).astype(jnp.bfloat16)
        ones = jnp.ones((1, r.shape[1]), jnp.bfloat16)
        r_aug = jnp.concatenate([r, ones], axis=0)
        out = jnp.dot(w2_ref[...], r_aug,
                      preferred_element_type=jnp.float32)
        o_ref[i] = out


def _position_embedding(xyz, w1, gamma, beta, w2, b2, *, tile_n=4096,
                        eps=1e-5):
    B, N, Cin = xyz.shape
    F = w1.shape[0]

    x_t = jnp.transpose(xyz.astype(jnp.bfloat16), (0, 2, 1))

    if N <= tile_n:
        tn, n_pad = N, N
    else:
        tn = tile_n
        n_pad = ((N + tn - 1) // tn) * tn
    if n_pad != N:
        x_t = jnp.pad(x_t, ((0, 0), (0, 0), (0, n_pad - N)))
    n_tiles = n_pad // tn
    x_t4 = jnp.concatenate(
        [x_t, jnp.ones((B, 1, n_pad), jnp.bfloat16)], axis=1)

    bb = next(d for d in (8, 4, 2, 1) if B % d == 0)
    g = B // bb
    parts = pl.pallas_call(
        _moments_kernel,
        out_shape=jax.ShapeDtypeStruct((g, Cin, Cin + 1), jnp.float32),
        grid=(g,),
        in_specs=[pl.BlockSpec((bb, Cin + 1, n_pad), lambda i: (i, 0, 0))],
        out_specs=pl.BlockSpec((1, Cin, Cin + 1), lambda i: (i, 0, 0)),
        compiler_params=pltpu.CompilerParams(
            dimension_semantics=("parallel",),
            vmem_limit_bytes=64 * 1024 * 1024),
    )(x_t4)
    mom = jnp.sum(parts, axis=0)

    P = B * N
    w1f = w1.astype(jnp.float32)
    mean_x = mom[:, 0] / P
    e_xxt = mom[:, 1:] / P
    cov_x = e_xxt - jnp.outer(mean_x, mean_x)
    mean_h = w1f @ mean_x
    var_h = jnp.maximum(jnp.einsum("fc,cd,fd->f", w1f, cov_x, w1f), 0.0)
    scale = gamma.astype(jnp.float32) * jax.lax.rsqrt(var_h + eps)
    shift = beta.astype(jnp.float32) - mean_h * scale
    w1a = jnp.concatenate(
        [w1f * scale[:, None], shift[:, None]],
        axis=1).astype(jnp.bfloat16)
    w2b = jnp.concatenate(
        [w2.astype(jnp.float32), b2.astype(jnp.float32)[:, None]],
        axis=1).astype(jnp.bfloat16)

    bb2 = 2 if B % 2 == 0 else 1
    out = pl.pallas_call(
        _apply_kernel,
        out_shape=jax.ShapeDtypeStruct((B, F, n_pad), jnp.float32),
        grid=(B // bb2, n_tiles),
        in_specs=[
            pl.BlockSpec((bb2, Cin + 1, tn), lambda b, t: (b, 0, t)),
            pl.BlockSpec((F, Cin + 1), lambda b, t: (0, 0)),
            pl.BlockSpec((F, F + 1), lambda b, t: (0, 0)),
        ],
        out_specs=pl.BlockSpec((bb2, F, tn), lambda b, t: (b, 0, t)),
        compiler_params=pltpu.CompilerParams(
            dimension_semantics=("parallel", "parallel"),
            vmem_limit_bytes=64 * 1024 * 1024),
    )(x_t4, w1a, w2b)

    if n_pad != N:
        out = out[:, :, :N]
    return out


def kernel(xyz, w1, b1, gamma, beta, w2, b2):
    del b1
    return _position_embedding(xyz, w1, gamma, beta, w2, b2, tile_n=16384)


# speedup vs baseline: 5.2793x; 1.0007x over previous
import jax
import jax.numpy as jnp
from jax.experimental import pallas as pl
from jax.experimental.pallas import tpu as pltpu


def _moments_kernel(x_ref, mom_ref):
    cin = x_ref.shape[1] - 1
    x = x_ref[:, :cin, :].astype(jnp.float32)
    s = jnp.sum(jnp.sum(x, axis=2, keepdims=True), axis=0)
    cols = [s]
    for c in range(cin):
        p = jnp.sum(x * x[:, c:c + 1, :], axis=2, keepdims=True)
        cols.append(jnp.sum(p, axis=0))
    mom_ref[...] = jnp.concatenate(cols, axis=1)[None]


def _apply_kernel(x_ref, w1_ref, w2_ref, o_ref):
    for i in range(x_ref.shape[0]):
        x16 = x_ref[i]
        h = jnp.dot(w1_ref[...], x16,
                    preferred_element_type=jnp.float32)
        r = jnp.maximum(h, 0.0).astype(jnp.bfloat16)
        ones = jnp.ones((1, r.shape[1]), jnp.bfloat16)
        r_aug = jnp.concatenate([r, ones], axis=0)
        out = jnp.dot(w2_ref[...], r_aug,
                      preferred_element_type=jnp.float32)
        o_ref[i] = out


def _position_embedding(xyz, w1, gamma, beta, w2, b2, *, tile_n=4096,
                        eps=1e-5):
    B, N, Cin = xyz.shape
    F = w1.shape[0]

    x_t = jnp.transpose(xyz.astype(jnp.bfloat16), (0, 2, 1))

    if N <= tile_n:
        tn, n_pad = N, N
    else:
        tn = tile_n
        n_pad = ((N + tn - 1) // tn) * tn
    if n_pad != N:
        x_t = jnp.pad(x_t, ((0, 0), (0, 0), (0, n_pad - N)))
    n_tiles = n_pad // tn
    x_t4 = jnp.concatenate(
        [x_t, jnp.ones((B, 1, n_pad), jnp.bfloat16)], axis=1)

    bb = next(d for d in (8, 4, 2, 1) if B % d == 0)
    g = B // bb
    parts = pl.pallas_call(
        _moments_kernel,
        out_shape=jax.ShapeDtypeStruct((g, Cin, Cin + 1), jnp.float32),
        grid=(g,),
        in_specs=[pl.BlockSpec((bb, Cin + 1, n_pad), lambda i: (i, 0, 0))],
        out_specs=pl.BlockSpec((1, Cin, Cin + 1), lambda i: (i, 0, 0)),
        compiler_params=pltpu.CompilerParams(
            dimension_semantics=("parallel",),
            vmem_limit_bytes=64 * 1024 * 1024),
    )(x_t4)
    mom = jnp.sum(parts, axis=0)

    P = B * N
    w1f = w1.astype(jnp.float32)
    mean_x = mom[:, 0] / P
    e_xxt = mom[:, 1:] / P
    cov_x = e_xxt - jnp.outer(mean_x, mean_x)
    mean_h = w1f @ mean_x
    var_h = jnp.maximum(jnp.einsum("fc,cd,fd->f", w1f, cov_x, w1f), 0.0)
    scale = gamma.astype(jnp.float32) * jax.lax.rsqrt(var_h + eps)
    shift = beta.astype(jnp.float32) - mean_h * scale
    w1a = jnp.concatenate(
        [w1f * scale[:, None], shift[:, None]],
        axis=1).astype(jnp.bfloat16)
    w2b = jnp.concatenate(
        [w2.astype(jnp.float32), b2.astype(jnp.float32)[:, None]],
        axis=1).astype(jnp.bfloat16)

    bb2 = 2 if B % 2 == 0 else 1
    out = pl.pallas_call(
        _apply_kernel,
        out_shape=jax.ShapeDtypeStruct((B, F, n_pad), jnp.float32),
        grid=(B // bb2, n_tiles),
        in_specs=[
            pl.BlockSpec((bb2, Cin + 1, tn), lambda b, t: (b, 0, t)),
            pl.BlockSpec((F, Cin + 1), lambda b, t: (0, 0)),
            pl.BlockSpec((F, F + 1), lambda b, t: (0, 0)),
        ],
        out_specs=pl.BlockSpec((bb2, F, tn), lambda b, t: (b, 0, t)),
        compiler_params=pltpu.CompilerParams(
            dimension_semantics=("parallel", "parallel"),
            vmem_limit_bytes=64 * 1024 * 1024),
    )(x_t4, w1a, w2b)

    if n_pad != N:
        out = out[:, :, :N]
    return out


def kernel(xyz, w1, b1, gamma, beta, w2, b2):
    del b1
    return _position_embedding(xyz, w1, gamma, beta, w2, b2, tile_n=16384)
